# baseline (speedup 1.0000x reference)
import functools

import jax
import jax.numpy as jnp
from jax import lax
from jax.experimental import pallas as pl
from jax.experimental.pallas import tpu as pltpu
from jax.experimental.pallas import tpu_sc as plsc

N_NODES = 50000
N_EDGES = 800000
N_TRIPLES = 1600000
DIM = 64
MAX_N = 3
MAX_L = 3
DEG = 9
DEGP = 16
UNITS = 64
CUTOFF = 5.0
TB_CUTOFF = 4.0

EBLK = 4000


def _silu(x):
    return x * jax.nn.sigmoid(x)


_SC_NC = 2
_SC_NW = 32
_LCH = 128
_NB = 7


@functools.lru_cache(maxsize=None)
def _sc_gather_fn(V, D, B):
    n_chunks = B // _LCH
    base_c = n_chunks // _SC_NW
    extra = n_chunks - base_c * _SC_NW
    maxc = base_c + (1 if extra else 0)
    mesh = plsc.VectorSubcoreMesh(core_axis_name="c", subcore_axis_name="s")

    @functools.partial(
        pl.kernel, mesh=mesh,
        out_type=jax.ShapeDtypeStruct((B, D), jnp.float32),
        scratch_types=(
            [pltpu.VMEM((_NB, _LCH), jnp.int32),
             pltpu.VMEM((_NB, _LCH, D), jnp.float32)]
            + [pltpu.SemaphoreType.DMA] * (3 * _NB)),
    )
    def k(table_h, idx_h, out_h, idx_v, rows_v, *sems):
        isem = sems[:_NB]
        gsem = sems[_NB:2 * _NB]
        osem = sems[2 * _NB:]
        s = lax.axis_index("s")
        c = lax.axis_index("c")
        w = s * _SC_NC + c
        c0 = w * base_c + jnp.minimum(w, extra)
        nw = base_c + jnp.where(w < extra, 1, 0)

        def i_desc(j, b):
            return pltpu.make_async_copy(
                idx_h.at[pl.ds((c0 + j) * _LCH, _LCH)], idx_v.at[b], isem[b])

        def g_desc(j, b):
            return pltpu.make_async_copy(
                table_h.at[idx_v.at[b]], rows_v.at[b], gsem[b])

        def o_desc(j, b):
            return pltpu.make_async_copy(
                rows_v.at[b], out_h.at[pl.ds((c0 + j) * _LCH, _LCH)], osem[b])

        for b in range(_NB):
            @pl.when(b < nw)
            def _(b=b):
                i_desc(b, b).start()
        for b in range(_NB):
            @pl.when(b < nw)
            def _(b=b):
                i_desc(b, b).wait()
                g_desc(b, b).start()

        n_outer = (maxc + _NB - 1) // _NB

        def outer(g, _):
            j0 = g * _NB
            for b in range(_NB):
                j = j0 + b
                jn = j + _NB

                @pl.when(j < nw)
                def _(j=j, jn=jn, b=b):
                    g_desc(j, b).wait()
                    o_desc(j, b).start()

                    @pl.when(jn < nw)
                    def _():
                        i_desc(jn, b).start()
                        o_desc(j, b).wait()
                        i_desc(jn, b).wait()
                        g_desc(jn, b).start()
            return ()

        lax.fori_loop(0, n_outer, outer, (), unroll=False)
        for b in range(_NB):
            @pl.when(b < nw)
            def _(b=b):
                o_desc(0, b).wait()

    return k


_SCH = 64


@functools.lru_cache(maxsize=None)
def _sc_scatter_fn(B, R, NPASS):
    D = 128
    n_chunks = B // _SCH
    base_t = n_chunks // 16
    extra_t = n_chunks - base_t * 16
    maxc = base_t + (1 if extra_t else 0)
    stripe = R // 16
    NBS = 2
    mesh = plsc.VectorSubcoreMesh(core_axis_name="c", subcore_axis_name="s")

    @functools.partial(
        pl.kernel, mesh=mesh,
        out_type=jax.ShapeDtypeStruct((2 * NPASS * R, D), jnp.float32),
        scratch_types=(
            [pltpu.VMEM_SHARED((R + _SCH, D), jnp.float32),
             pltpu.VMEM((NBS, _SCH, D), jnp.float32),
             pltpu.VMEM((NBS, _SCH), jnp.int32)]
            + [pltpu.SemaphoreType.DMA] * (3 * NBS)),
    )
    def k(data_h, idx_h, zeros_h, out_h, acc, dbuf, ibuf, *sems):
        dsem = sems[:NBS]
        isem = sems[NBS:2 * NBS]
        asem = sems[2 * NBS:]
        s = lax.axis_index("s")
        c = lax.axis_index("c")
        t0 = s * base_t + jnp.minimum(s, extra_t)
        nt = base_t + jnp.where(s < extra_t, 1, 0)

        def d_desc(j, b):
            return pltpu.make_async_copy(
                data_h.at[pl.ds((t0 + j) * _SCH, _SCH)], dbuf.at[b], dsem[b])

        def i_desc(j, b):
            return pltpu.make_async_copy(
                idx_h.at[pl.ds((t0 + j) * _SCH, _SCH)], ibuf.at[b], isem[b])

        def a_desc(b):
            return pltpu.make_async_copy(dbuf.at[b], acc.at[ibuf.at[b]], asem[b])

        for p in range(NPASS):
            rid = 2 * p + c
            lo = rid * R
            pltpu.sync_copy(zeros_h, acc.at[pl.ds(s * stripe, stripe)])
            plsc.subcore_barrier()

            for b in range(NBS):
                @pl.when(b < nt)
                def _(b=b):
                    d_desc(b, b).start()
                    i_desc(b, b).start()

            def outer(g, _):
                j0 = g * NBS
                for b in range(NBS):
                    j = j0 + b
                    jn = j + NBS

                    @pl.when(j < nt)
                    def _(j=j, jn=jn, b=b):
                        d_desc(j, b).wait()
                        i_desc(j, b).wait()
                        for v in range(_SCH // 16):
                            x = ibuf[b, pl.ds(v * 16, 16)]
                            ok = jnp.logical_and(x >= lo, x < lo + R)
                            ibuf[b, pl.ds(v * 16, 16)] = jnp.where(ok, x - lo, R)
                        a_desc(b).start(add=True)

                        @pl.when(jn < nt)
                        def _():
                            a_desc(b).wait()
                            d_desc(jn, b).start()
                            i_desc(jn, b).start()
                return ()

            lax.fori_loop(0, (maxc + NBS - 1) // NBS, outer, (), unroll=False)
            for b in range(NBS):
                @pl.when(b < nt)
                def _(b=b):
                    a_desc(b).wait()
            plsc.subcore_barrier()
            pltpu.sync_copy(acc.at[pl.ds(s * stripe, stripe)],
                            out_h.at[pl.ds(rid * R + s * stripe, stripe)])
            plsc.subcore_barrier()

    return k


def _sc_segsum(data, idx, n_out, R, npass):
    B = data.shape[0]
    zeros = jnp.zeros((R // 16, 128), jnp.float32)
    out = _sc_scatter_fn(B, R, npass)(data, idx.astype(jnp.int32), zeros)
    return out[:n_out]


def _sc_gather(table, idx, d_out=None):
    V, D = table.shape
    if D != 128:
        table = jnp.pad(table, ((0, 0), (0, 128 - D)))
    B0 = idx.shape[0]
    B = ((B0 + _LCH - 1) // _LCH) * _LCH
    if B != B0:
        idx = jnp.pad(idx, (0, B - B0))
    out = _sc_gather_fn(V, 128, B)(table, idx.astype(jnp.int32))
    if B != B0:
        out = out[:B0]
    return out if d_out is None else out[:, :d_out]


def _block_edge_kernel(agg_ref, sf_ref, df_ref, ef_ref, rbf_ref,
                       tbW_ref, eA1_ref, eb1_ref, eA2_ref, eb2_ref, eWr_ref, ebr_ref,
                       nA1_ref, nb1_ref, nA2_ref, nb2_ref, nWr_ref, nbr_ref,
                       efo_ref, msg_ref):
    a = agg_ref[...]
    t = jnp.dot(a, tbW_ref[...], preferred_element_type=jnp.float32)
    ef = ef_ref[...] + _silu(t[:, :DIM]) * jax.nn.sigmoid(t[:, DIM:])
    s = sf_ref[...][:, :DIM]
    d = df_ref[...][:, :DIM]
    rbf = rbf_ref[...]

    def gated(x_s, x_d, x_e, A1, b1, A2, b2, Wr, br):
        x1 = (jnp.dot(x_s, A1[:DIM], preferred_element_type=jnp.float32)
              + jnp.dot(x_d, A1[DIM:2 * DIM], preferred_element_type=jnp.float32)
              + jnp.dot(x_e, A1[2 * DIM:], preferred_element_type=jnp.float32)
              + b1[None, :])
        u = _silu(x1)
        y = jnp.dot(u, A2, preferred_element_type=jnp.float32) + b2[None, :]
        h = _silu(y[:, :UNITS])
        g = jax.nn.sigmoid(y[:, UNITS:])
        r = jnp.dot(rbf, Wr, preferred_element_type=jnp.float32) + br[None, :]
        return h * g * r

    ef2 = ef + gated(s, d, ef, eA1_ref[...], eb1_ref[...], eA2_ref[...],
                     eb2_ref[...], eWr_ref[...], ebr_ref[...])
    efo_ref[...] = ef2
    mg = gated(s, d, ef2, nA1_ref[...], nb1_ref[...], nA2_ref[...],
               nb2_ref[...], nWr_ref[...], nbr_ref[...])
    msg_ref[...] = jnp.concatenate([mg, jnp.zeros_like(mg)], axis=1)


def _run_block_edge(agg, nf2, ef, rbf, w):
    E = ef.shape[0]
    grid = (E // EBLK,)
    neb = E // EBLK
    row = lambda i: (i, 0)
    full = lambda i: (0, 0)
    vec = lambda i: (0,)
    in_specs = [
        pl.BlockSpec((EBLK, DEGP), row),
        pl.BlockSpec((EBLK, 128), row),
        pl.BlockSpec((EBLK, 128), lambda i: (i + neb, 0)),
        pl.BlockSpec((EBLK, DIM), row),
        pl.BlockSpec((EBLK, DEGP), row),
        pl.BlockSpec((DEGP, 2 * DIM), full),
    ]
    for _ in range(2):
        in_specs += [
            pl.BlockSpec((3 * DIM, 2 * UNITS), full),
            pl.BlockSpec((2 * UNITS,), vec),
            pl.BlockSpec((2 * UNITS, 2 * UNITS), full),
            pl.BlockSpec((2 * UNITS,), vec),
            pl.BlockSpec((DEGP, UNITS), full),
            pl.BlockSpec((UNITS,), vec),
        ]
    out_specs = [
        pl.BlockSpec((EBLK, DIM), row),
        pl.BlockSpec((EBLK, 2 * DIM), row),
    ]
    return pl.pallas_call(
        _block_edge_kernel,
        grid=grid,
        in_specs=in_specs,
        out_specs=out_specs,
        out_shape=[
            jax.ShapeDtypeStruct((E, DIM), jnp.float32),
            jax.ShapeDtypeStruct((E, 2 * DIM), jnp.float32),
        ],
        compiler_params=pltpu.CompilerParams(
            dimension_semantics=("arbitrary",)),
    )(agg, nf2, nf2, ef, rbf, *w)


def _pack_block_weights(b):
    tbW = jnp.concatenate([b['tb_Wh'], b['tb_Wg']], axis=1)
    tbW = jnp.pad(tbW, ((0, DEGP - DEG), (0, 0)))
    out = [tbW]
    for pfx in ('e', 'n'):
        A1 = jnp.concatenate([b[pfx + '_W1'], b[pfx + '_Wg1']], axis=1)
        b1 = jnp.concatenate([b[pfx + '_b1'], b[pfx + '_bg1']])
        z = jnp.zeros((UNITS, UNITS), jnp.float32)
        A2 = jnp.concatenate([
            jnp.concatenate([b[pfx + '_W2'], z], axis=1),
            jnp.concatenate([z, b[pfx + '_Wg2']], axis=1)], axis=0)
        b2 = jnp.concatenate([b[pfx + '_b2'], b[pfx + '_bg2']])
        Wr = jnp.pad(b[pfx + '_Wr'], ((0, DEGP - DEG), (0, 0)))
        out += [A1, b1, A2, b2, Wr, b[pfx + '_br']]
    return out


def _poly_cutoff(r, rc):
    x = r / rc
    f = 1.0 - 6.0 * x**5 + 15.0 * x**4 - 10.0 * x**3
    return jnp.where(r < rc, f, jnp.zeros_like(f))


def kernel(node_pos, node_type, edge_index, triple_edges, triple_center, params):
    src, dst = edge_index[0], edge_index[1]
    E = src.shape[0]
    sd = jnp.concatenate([src, dst])
    gpos = _sc_gather(node_pos, sd, 3)
    bond_vec = gpos[E:] - gpos[:E]
    bond_dist = jnp.sqrt(jnp.sum(bond_vec**2, axis=-1) + 1e-12)
    centers = jnp.linspace(0.0, CUTOFF, DEG)
    rbf = jnp.exp(-((bond_dist[:, None] - centers[None, :]) / 0.5) ** 2)
    rbf_p = jnp.pad(rbf, ((0, 0), (0, DEGP - DEG)))
    tb_cut = _poly_cutoff(bond_dist, TB_CUTOFF)

    ei, ej = triple_edges[0], triple_edges[1]
    T = ei.shape[0]
    vpack = jnp.concatenate(
        [bond_vec, bond_dist[:, None], tb_cut[:, None]], axis=1)
    gij = _sc_gather(vpack, jnp.concatenate([ei, ej]), 5)
    gi, gj = gij[:T], gij[T:]
    vi, vj = gi[:, :3], gj[:, :3]
    ri, rj = gi[:, 3], gj[:, 3]
    cos_t = jnp.sum(vi * vj, axis=-1) / (ri * rj + 1e-12)
    cos_t = jnp.clip(cos_t, -1.0, 1.0)
    n_idx = jnp.arange(1, MAX_N + 1, dtype=jnp.float32)
    sbf = jnp.sin(n_idx[None, :] * jnp.pi * ri[:, None] / TB_CUTOFF) / (ri[:, None] + 1e-8)
    shf = jnp.stack([jnp.ones_like(cos_t), cos_t, 0.5 * (3.0 * cos_t**2 - 1.0)], axis=-1)
    basis = (sbf[:, :, None] * shf[:, None, :]).reshape(-1, DEG)
    bc = basis * (gi[:, 4] * gj[:, 4])[:, None]

    node_feat = _sc_gather(params['node_embed'], node_type, DIM)
    edge_feat = jax.nn.silu(rbf @ params['edge_W'] + params['edge_b'])

    for b in params['blocks']:
        atom_w = jax.nn.sigmoid(node_feat @ b['tb_Wa'] + b['tb_ba'])
        m = bc * _sc_gather(atom_w, triple_center, DEG)
        agg = jax.ops.segment_sum(m, ej, num_segments=N_EDGES)
        agg_p = jnp.pad(agg, ((0, 0), (0, DEGP - DEG)))
        w = _pack_block_weights(b)
        edge_feat, msg = _run_block_edge(agg_p, _sc_gather(node_feat, sd),
                                         edge_feat, rbf_p, w)
        seg = _sc_segsum(msg, dst, N_NODES, 13312, 2)
        node_feat = node_feat + seg[:, :DIM]

    ro = params['readout']
    wgt = jax.nn.sigmoid(node_feat @ ro['Wg'] + ro['bg'])
    h = jax.nn.silu(node_feat @ ro['W1'] + ro['b1'])
    h = jax.nn.silu(h @ ro['W2'] + ro['b2'])
    g = jnp.sum(wgt * h, axis=0, keepdims=True)
    fi = params['final']
    out = jax.nn.silu(g @ fi['W1'] + fi['b1'])
    out = jax.nn.silu(out @ fi['W2'] + fi['b2'])
    out = out @ fi['W3'] + fi['b3']
    return out


# speedup vs baseline: 1.0153x; 1.0153x over previous
import functools

import jax
import jax.numpy as jnp
from jax import lax
from jax.experimental import pallas as pl
from jax.experimental.pallas import tpu as pltpu
from jax.experimental.pallas import tpu_sc as plsc

N_NODES = 50000
N_EDGES = 800000
N_TRIPLES = 1600000
DIM = 64
MAX_N = 3
MAX_L = 3
DEG = 9
DEGP = 16
UNITS = 64
CUTOFF = 5.0
TB_CUTOFF = 4.0

EBLK = 4000


def _silu(x):
    return x * jax.nn.sigmoid(x)


_SC_NC = 2
_SC_NW = 32
_LCH = 128
_NB = 7


@functools.lru_cache(maxsize=None)
def _sc_gather_fn(V, D, B):
    n_chunks = B // _LCH
    base_c = n_chunks // _SC_NW
    extra = n_chunks - base_c * _SC_NW
    maxc = base_c + (1 if extra else 0)
    mesh = plsc.VectorSubcoreMesh(core_axis_name="c", subcore_axis_name="s")

    @functools.partial(
        pl.kernel, mesh=mesh,
        out_type=jax.ShapeDtypeStruct((B, D), jnp.float32),
        scratch_types=(
            [pltpu.VMEM((_NB, _LCH), jnp.int32),
             pltpu.VMEM((_NB, _LCH, D), jnp.float32)]
            + [pltpu.SemaphoreType.DMA] * (3 * _NB)),
    )
    def k(table_h, idx_h, out_h, idx_v, rows_v, *sems):
        isem = sems[:_NB]
        gsem = sems[_NB:2 * _NB]
        osem = sems[2 * _NB:]
        s = lax.axis_index("s")
        c = lax.axis_index("c")
        w = s * _SC_NC + c
        c0 = w * base_c + jnp.minimum(w, extra)
        nw = base_c + jnp.where(w < extra, 1, 0)

        def i_desc(j, b):
            return pltpu.make_async_copy(
                idx_h.at[pl.ds((c0 + j) * _LCH, _LCH)], idx_v.at[b], isem[b])

        def g_desc(j, b):
            return pltpu.make_async_copy(
                table_h.at[idx_v.at[b]], rows_v.at[b], gsem[b])

        def o_desc(j, b):
            return pltpu.make_async_copy(
                rows_v.at[b], out_h.at[pl.ds((c0 + j) * _LCH, _LCH)], osem[b])

        for b in range(_NB):
            @pl.when(b < nw)
            def _(b=b):
                i_desc(b, b).start()
        for b in range(_NB):
            @pl.when(b < nw)
            def _(b=b):
                i_desc(b, b).wait()
                g_desc(b, b).start()

        n_outer = (maxc + _NB - 1) // _NB

        def outer(g, _):
            j0 = g * _NB
            for b in range(_NB):
                j = j0 + b
                jn = j + _NB

                @pl.when(j < nw)
                def _(j=j, jn=jn, b=b):
                    g_desc(j, b).wait()
                    o_desc(j, b).start()

                    @pl.when(jn < nw)
                    def _():
                        i_desc(jn, b).start()
                        o_desc(j, b).wait()
                        i_desc(jn, b).wait()
                        g_desc(jn, b).start()
            return ()

        lax.fori_loop(0, n_outer, outer, (), unroll=False)
        for b in range(_NB):
            @pl.when(b < nw)
            def _(b=b):
                o_desc(0, b).wait()

    return k


def _sc_gather(table, idx, d_out=None):
    V, D = table.shape
    if D != 128:
        table = jnp.pad(table, ((0, 0), (0, 128 - D)))
    B0 = idx.shape[0]
    B = ((B0 + _LCH - 1) // _LCH) * _LCH
    if B != B0:
        idx = jnp.pad(idx, (0, B - B0))
    out = _sc_gather_fn(V, 128, B)(table, idx.astype(jnp.int32))
    if B != B0:
        out = out[:B0]
    return out if d_out is None else out[:, :d_out]


def _block_edge_kernel(agg_ref, sf_ref, df_ref, ef_ref, rbf_ref,
                       tbW_ref, eA1_ref, eb1_ref, eA2_ref, eb2_ref, eWr_ref, ebr_ref,
                       nA1_ref, nb1_ref, nA2_ref, nb2_ref, nWr_ref, nbr_ref,
                       efo_ref, msg_ref):
    a = agg_ref[...]
    t = jnp.dot(a, tbW_ref[...], preferred_element_type=jnp.float32)
    ef = ef_ref[...] + _silu(t[:, :DIM]) * jax.nn.sigmoid(t[:, DIM:])
    s = sf_ref[...][:, :DIM]
    d = df_ref[...][:, :DIM]
    rbf = rbf_ref[...]

    def gated(x_s, x_d, x_e, A1, b1, A2, b2, Wr, br):
        x1 = (jnp.dot(x_s, A1[:DIM], preferred_element_type=jnp.float32)
              + jnp.dot(x_d, A1[DIM:2 * DIM], preferred_element_type=jnp.float32)
              + jnp.dot(x_e, A1[2 * DIM:], preferred_element_type=jnp.float32)
              + b1[None, :])
        u = _silu(x1)
        y = jnp.dot(u, A2, preferred_element_type=jnp.float32) + b2[None, :]
        h = _silu(y[:, :UNITS])
        g = jax.nn.sigmoid(y[:, UNITS:])
        r = jnp.dot(rbf, Wr, preferred_element_type=jnp.float32) + br[None, :]
        return h * g * r

    ef2 = ef + gated(s, d, ef, eA1_ref[...], eb1_ref[...], eA2_ref[...],
                     eb2_ref[...], eWr_ref[...], ebr_ref[...])
    efo_ref[...] = ef2
    msg_ref[...] = gated(s, d, ef2, nA1_ref[...], nb1_ref[...], nA2_ref[...],
                         nb2_ref[...], nWr_ref[...], nbr_ref[...])


def _run_block_edge(agg, nf2, ef, rbf, w):
    E = ef.shape[0]
    grid = (E // EBLK,)
    neb = E // EBLK
    row = lambda i: (i, 0)
    full = lambda i: (0, 0)
    vec = lambda i: (0,)
    in_specs = [
        pl.BlockSpec((EBLK, DEGP), row),
        pl.BlockSpec((EBLK, 128), row),
        pl.BlockSpec((EBLK, 128), lambda i: (i + neb, 0)),
        pl.BlockSpec((EBLK, DIM), row),
        pl.BlockSpec((EBLK, DEGP), row),
        pl.BlockSpec((DEGP, 2 * DIM), full),
    ]
    for _ in range(2):
        in_specs += [
            pl.BlockSpec((3 * DIM, 2 * UNITS), full),
            pl.BlockSpec((2 * UNITS,), vec),
            pl.BlockSpec((2 * UNITS, 2 * UNITS), full),
            pl.BlockSpec((2 * UNITS,), vec),
            pl.BlockSpec((DEGP, UNITS), full),
            pl.BlockSpec((UNITS,), vec),
        ]
    out_specs = [
        pl.BlockSpec((EBLK, DIM), row),
        pl.BlockSpec((EBLK, DIM), row),
    ]
    return pl.pallas_call(
        _block_edge_kernel,
        grid=grid,
        in_specs=in_specs,
        out_specs=out_specs,
        out_shape=[
            jax.ShapeDtypeStruct((E, DIM), jnp.float32),
            jax.ShapeDtypeStruct((E, DIM), jnp.float32),
        ],
        compiler_params=pltpu.CompilerParams(
            dimension_semantics=("arbitrary",)),
    )(agg, nf2, nf2, ef, rbf, *w)


def _pack_block_weights(b):
    tbW = jnp.concatenate([b['tb_Wh'], b['tb_Wg']], axis=1)
    tbW = jnp.pad(tbW, ((0, DEGP - DEG), (0, 0)))
    out = [tbW]
    for pfx in ('e', 'n'):
        A1 = jnp.concatenate([b[pfx + '_W1'], b[pfx + '_Wg1']], axis=1)
        b1 = jnp.concatenate([b[pfx + '_b1'], b[pfx + '_bg1']])
        z = jnp.zeros((UNITS, UNITS), jnp.float32)
        A2 = jnp.concatenate([
            jnp.concatenate([b[pfx + '_W2'], z], axis=1),
            jnp.concatenate([z, b[pfx + '_Wg2']], axis=1)], axis=0)
        b2 = jnp.concatenate([b[pfx + '_b2'], b[pfx + '_bg2']])
        Wr = jnp.pad(b[pfx + '_Wr'], ((0, DEGP - DEG), (0, 0)))
        out += [A1, b1, A2, b2, Wr, b[pfx + '_br']]
    return out


def _poly_cutoff(r, rc):
    x = r / rc
    f = 1.0 - 6.0 * x**5 + 15.0 * x**4 - 10.0 * x**3
    return jnp.where(r < rc, f, jnp.zeros_like(f))


def kernel(node_pos, node_type, edge_index, triple_edges, triple_center, params):
    src, dst = edge_index[0], edge_index[1]
    E = src.shape[0]
    sd = jnp.concatenate([src, dst])
    gpos = _sc_gather(node_pos, sd, 3)
    bond_vec = gpos[E:] - gpos[:E]
    bond_dist = jnp.sqrt(jnp.sum(bond_vec**2, axis=-1) + 1e-12)
    centers = jnp.linspace(0.0, CUTOFF, DEG)
    rbf = jnp.exp(-((bond_dist[:, None] - centers[None, :]) / 0.5) ** 2)
    rbf_p = jnp.pad(rbf, ((0, 0), (0, DEGP - DEG)))
    tb_cut = _poly_cutoff(bond_dist, TB_CUTOFF)

    ei, ej = triple_edges[0], triple_edges[1]
    T = ei.shape[0]
    vpack = jnp.concatenate(
        [bond_vec, bond_dist[:, None], tb_cut[:, None]], axis=1)
    gij = _sc_gather(vpack, jnp.concatenate([ei, ej]), 5)
    gi, gj = gij[:T], gij[T:]
    vi, vj = gi[:, :3], gj[:, :3]
    ri, rj = gi[:, 3], gj[:, 3]
    cos_t = jnp.sum(vi * vj, axis=-1) / (ri * rj + 1e-12)
    cos_t = jnp.clip(cos_t, -1.0, 1.0)
    n_idx = jnp.arange(1, MAX_N + 1, dtype=jnp.float32)
    sbf = jnp.sin(n_idx[None, :] * jnp.pi * ri[:, None] / TB_CUTOFF) / (ri[:, None] + 1e-8)
    shf = jnp.stack([jnp.ones_like(cos_t), cos_t, 0.5 * (3.0 * cos_t**2 - 1.0)], axis=-1)
    basis = (sbf[:, :, None] * shf[:, None, :]).reshape(-1, DEG)
    bc = basis * (gi[:, 4] * gj[:, 4])[:, None]

    node_feat = _sc_gather(params['node_embed'], node_type, DIM)
    edge_feat = jax.nn.silu(rbf @ params['edge_W'] + params['edge_b'])

    for b in params['blocks']:
        atom_w = jax.nn.sigmoid(node_feat @ b['tb_Wa'] + b['tb_ba'])
        m = bc * _sc_gather(atom_w, triple_center, DEG)
        agg = jax.ops.segment_sum(m, ej, num_segments=N_EDGES)
        agg_p = jnp.pad(agg, ((0, 0), (0, DEGP - DEG)))
        w = _pack_block_weights(b)
        edge_feat, msg = _run_block_edge(agg_p, _sc_gather(node_feat, sd),
                                         edge_feat, rbf_p, w)
        node_feat = node_feat + jax.ops.segment_sum(msg, dst, num_segments=N_NODES)

    ro = params['readout']
    wgt = jax.nn.sigmoid(node_feat @ ro['Wg'] + ro['bg'])
    h = jax.nn.silu(node_feat @ ro['W1'] + ro['b1'])
    h = jax.nn.silu(h @ ro['W2'] + ro['b2'])
    g = jnp.sum(wgt * h, axis=0, keepdims=True)
    fi = params['final']
    out = jax.nn.silu(g @ fi['W1'] + fi['b1'])
    out = jax.nn.silu(out @ fi['W2'] + fi['b2'])
    out = out @ fi['W3'] + fi['b3']
    return out
